# baseline (speedup 1.0000x reference)
import jax
import jax.numpy as jnp
from jax.experimental import pallas as pl
from jax.experimental.pallas import tpu as pltpu

_RB = 16


def _upconv_kernel(xa_ref, xb_ref, w0_ref, w1_ref, g_ref, b_ref, o_ref):
    cin = xa_ref.shape[1]
    wp2 = xa_ref.shape[2]
    cout = o_ref.shape[0]

    x10 = jnp.concatenate(
        [xa_ref[...].reshape(_RB * cin, wp2),
         xb_ref[0:2].reshape(2 * cin, wp2)], axis=0)

    a_kx = [
        jnp.dot(x10, g_ref[kx],
                preferred_element_type=jnp.float32).astype(jnp.bfloat16)
        for kx in range(3)
    ]

    w0 = w0_ref[...]
    w1 = w1_ref[...]
    bias = b_ref[...]

    for k in range(_RB):
        for py, w in ((0, w0), (1, w1)):
            base = (k + py) * cin
            y = bias
            for kx in range(3):
                y = y + jnp.dot(w[:, kx * 2 * cin:(kx + 1) * 2 * cin],
                                a_kx[kx][base:base + 2 * cin],
                                preferred_element_type=jnp.float32)
            o_ref[:, 2 * k + py, :] = y.astype(o_ref.dtype)


def kernel(x_nchw, weight, bias):
    n, cin, h, w = x_nchw.shape
    cout = weight.shape[0]
    ho, wo = 2 * h, 2 * w
    wp2 = w + 2
    nblk = h // _RB
    hp = (nblk + 1) * _RB

    x_pad = jnp.pad(x_nchw,
                    ((0, 0), (0, 0), (1, hp - h - 1), (1, 1))
                    ).astype(jnp.bfloat16)
    x_t = jnp.transpose(x_pad, (0, 2, 1, 3))

    def pack(wa, wb):
        t = jnp.stack([jnp.transpose(wa, (0, 2, 1)),
                       jnp.transpose(wb, (0, 2, 1))], axis=2)
        return t.reshape(cout, 6 * cin).astype(jnp.bfloat16)

    wk = weight
    w0_mat = pack(wk[:, :, 0, :], wk[:, :, 1, :] + wk[:, :, 2, :])
    w1_mat = pack(wk[:, :, 0, :] + wk[:, :, 1, :], wk[:, :, 2, :])

    ow_idx = jnp.arange(wo)
    g_list = []
    for kx in range(3):
        j = ow_idx + kx
        src = jnp.where(j == 0, 0,
                        jnp.where(j == wo + 1, w + 1, (j - 1) // 2 + 1))
        g_list.append(jnp.arange(wp2)[:, None] == src[None, :])
    g_all = jnp.stack(g_list, axis=0).astype(jnp.bfloat16)

    bias2d = bias.reshape(cout, 1)

    return pl.pallas_call(
        _upconv_kernel,
        out_shape=jax.ShapeDtypeStruct((n, cout, ho, wo), x_nchw.dtype),
        grid_spec=pltpu.PrefetchScalarGridSpec(
            num_scalar_prefetch=0,
            grid=(n, nblk),
            in_specs=[
                pl.BlockSpec((None, _RB, cin, wp2), lambda b, r: (b, r, 0, 0)),
                pl.BlockSpec((None, _RB, cin, wp2),
                             lambda b, r: (b, r + 1, 0, 0)),
                pl.BlockSpec((cout, 6 * cin), lambda b, r: (0, 0)),
                pl.BlockSpec((cout, 6 * cin), lambda b, r: (0, 0)),
                pl.BlockSpec((3, wp2, wo), lambda b, r: (0, 0, 0)),
                pl.BlockSpec((cout, 1), lambda b, r: (0, 0)),
            ],
            out_specs=pl.BlockSpec((None, cout, 2 * _RB, wo),
                                   lambda b, r: (b, 0, r, 0)),
        ),
        compiler_params=pltpu.CompilerParams(
            dimension_semantics=("parallel", "parallel"),
            vmem_limit_bytes=100 * 1024 * 1024,
        ),
    )(x_t, x_t, w0_mat, w1_mat, g_all, bias2d)
